# Initial kernel scaffold; baseline (speedup 1.0000x reference)
import jax
import jax.numpy as jnp
from jax.experimental import pallas as pl


def kernel(x, edge_index, W0, b0, W1, b1, W2, b2, Wf, bf):
    raise NotImplementedError("write your pallas kernel here")


# speedup vs baseline: 19.0474x; 19.0474x over previous
import functools

import jax
import jax.numpy as jnp
from jax import lax
from jax.experimental import pallas as pl
from jax.experimental.pallas import tpu as pltpu
from jax.experimental.pallas import tpu_sc as plsc

NPAD = 10240
D = 128
CH = 80
NCH = 125
ROWS_PER_TILE = 640

_mesh = plsc.VectorSubcoreMesh(core_axis_name="c", subcore_axis_name="s")


@functools.partial(
    pl.kernel,
    out_type=jax.ShapeDtypeStruct((2, NPAD), jnp.float32),
    mesh=_mesh,
    scratch_types=[
        pltpu.VMEM((NCH, CH), jnp.int32),
        pltpu.VMEM((CH,), jnp.float32),
        pltpu.VMEM_SHARED((NPAD,), jnp.float32),
    ],
)
def _deg_kernel(col_ref, ones_ref, z1_ref, out_ref, col_v, ones_v, deg_sh):
    cid = lax.axis_index("c")
    sid = lax.axis_index("s")
    pltpu.sync_copy(ones_ref, ones_v)
    pltpu.sync_copy(col_ref.at[cid * 16 + sid], col_v)
    sl = pl.ds(sid * ROWS_PER_TILE, ROWS_PER_TILE)
    pltpu.sync_copy(z1_ref.at[sl], deg_sh.at[sl])
    plsc.subcore_barrier()

    def body(i, carry):
        pltpu.sync_copy(ones_v, deg_sh.at[col_v.at[i]], add=True)
        return carry

    lax.fori_loop(0, NCH, body, 0)
    plsc.subcore_barrier()
    pltpu.sync_copy(deg_sh.at[sl], out_ref.at[cid, sl])


@functools.partial(
    pl.kernel,
    out_type=jax.ShapeDtypeStruct((2, NPAD, D), jnp.float32),
    mesh=_mesh,
    scratch_types=[
        pltpu.VMEM((NCH, CH), jnp.int32),
        pltpu.VMEM((NCH, CH), jnp.int32),
        pltpu.VMEM((CH, D), jnp.float32),
        pltpu.VMEM_SHARED((NPAD, D), jnp.float32),
        pltpu.SemaphoreType.DMA,
    ],
)
def _hop_kernel(row_ref, col_ref, g_ref, z2_ref, out_ref,
                row_v, col_v, rows_v, acc_sh, sem):
    cid = lax.axis_index("c")
    sid = lax.axis_index("s")
    wid = cid * 16 + sid
    pltpu.sync_copy(row_ref.at[wid], row_v)
    pltpu.sync_copy(col_ref.at[wid], col_v)
    sl = pl.ds(sid * ROWS_PER_TILE, ROWS_PER_TILE)
    pltpu.sync_copy(z2_ref.at[sl], acc_sh.at[sl])
    plsc.subcore_barrier()

    def body(i, carry):
        pltpu.async_copy(g_ref.at[row_v.at[i]], rows_v, sem).wait()
        pltpu.sync_copy(rows_v, acc_sh.at[col_v.at[i]], add=True)
        return carry

    lax.fori_loop(0, NCH, body, 0)
    plsc.subcore_barrier()
    pltpu.sync_copy(acc_sh.at[sl], out_ref.at[cid, sl])


_BN = 512
_GRID = NPAD // _BN


def _tc1_body(d_ref, x_ref, o_ref):
    cnt = d_ref[0, :] + d_ref[1, :] + 1.0
    dis = lax.rsqrt(cnt)
    o_ref[...] = x_ref[...] * dis[:, None]


def _tc2_body(d_ref, s_ref, g_ref, o_ref):
    cnt = d_ref[0, :] + d_ref[1, :] + 1.0
    inv = 1.0 / cnt
    o_ref[...] = (s_ref[0] + s_ref[1] + g_ref[...]) * inv[:, None]


def _tc3_body(d_ref, x_ref, g2_ref, s2_ref, w0_ref, b0_ref, w1_ref, b1_ref,
              w2_ref, b2_ref, wf_ref, bf_ref, o_ref):
    cnt = d_ref[0, :] + d_ref[1, :] + 1.0
    dis = lax.rsqrt(cnt)
    sq = jnp.sqrt(cnt)
    g2 = g2_ref[...]
    x1 = g2 * sq[:, None]
    x2 = (s2_ref[0] + s2_ref[1] + g2) * dis[:, None]
    dot = functools.partial(jnp.dot, preferred_element_type=jnp.float32)
    t0 = jnp.maximum(dot(x_ref[...], w0_ref[...]) + b0_ref[...], 0.0)
    t1 = jnp.maximum(dot(x1, w1_ref[...]) + b1_ref[...], 0.0)
    t2 = jnp.maximum(dot(x2, w2_ref[...]) + b2_ref[...], 0.0)
    wf = wf_ref[...]
    out = dot(t0, wf[:64]) + dot(t1, wf[64:128]) + dot(t2, wf[128:]) \
        + bf_ref[...]
    o_ref[...] = out


def _spec_d():
    return pl.BlockSpec((2, _BN), lambda i: (0, i))


def _spec_rows():
    return pl.BlockSpec((_BN, D), lambda i: (i, 0))


def _spec_pair():
    return pl.BlockSpec((2, _BN, D), lambda i: (0, i, 0))


def _spec_full(shape):
    return pl.BlockSpec(shape, lambda i: tuple(0 for _ in shape))


def kernel(x, edge_index, W0, b0, W1, b1, W2, b2, Wf, bf):
    n, d = x.shape
    f32 = jnp.float32

    x_pad = jnp.pad(x, ((0, NPAD - n), (0, 0)))
    row_rs = edge_index[0].reshape(32, NCH, CH)
    col_rs = edge_index[1].reshape(32, NCH, CH)
    z1 = jnp.zeros((NPAD,), f32)
    z2 = jnp.zeros((NPAD, D), f32)
    ones = jnp.ones((CH,), f32)

    deg2 = _deg_kernel(col_rs, ones, z1)

    g1 = pl.pallas_call(
        _tc1_body,
        grid=(_GRID,),
        in_specs=[_spec_d(), _spec_rows()],
        out_specs=_spec_rows(),
        out_shape=jax.ShapeDtypeStruct((NPAD, D), f32),
    )(deg2, x_pad)

    s1p = _hop_kernel(row_rs, col_rs, g1, z2)

    g2 = pl.pallas_call(
        _tc2_body,
        grid=(_GRID,),
        in_specs=[_spec_d(), _spec_pair(), _spec_rows()],
        out_specs=_spec_rows(),
        out_shape=jax.ShapeDtypeStruct((NPAD, D), f32),
    )(deg2, s1p, g1)

    s2p = _hop_kernel(row_rs, col_rs, g2, z2)

    out = pl.pallas_call(
        _tc3_body,
        grid=(_GRID,),
        in_specs=[
            _spec_d(), _spec_rows(), _spec_rows(), _spec_pair(),
            _spec_full((D, 64)), _spec_full((1, 64)),
            _spec_full((D, 64)), _spec_full((1, 64)),
            _spec_full((D, 64)), _spec_full((1, 64)),
            _spec_full((192, D)), _spec_full((1, D)),
        ],
        out_specs=_spec_rows(),
        out_shape=jax.ShapeDtypeStruct((NPAD, D), f32),
    )(deg2, x_pad, g2, s2p, W0, b0.reshape(1, 64), W1, b1.reshape(1, 64),
      W2, b2.reshape(1, 64), Wf, bf.reshape(1, D))

    return out[:n]
